# baseline (speedup 1.0000x reference)
import functools

import jax
import jax.numpy as jnp
from jax import lax
from jax.experimental import pallas as pl
from jax.experimental.pallas import tpu as pltpu
from jax.experimental.pallas import tpu_sc as plsc

_N_NODES = 10000
_N_EDGES = 320000
_D = 128

_NC, _NS = 2, 16
_NW = _NC * _NS
_CH = 128
_EPAD = 327680
_NHALF = 2
_HALF_E = _EPAD // _NHALF
_CPT = _HALF_E // (_NW * _CH)
_EPT = _CPT * _CH
_NACC = _N_NODES + 16
_RPT = 624
_L = 16


_EB = 10240


_DW = _D + 16


def _payload_body(eat_ref, w1te_ref, b1_ref, out_ref):
    aT = eat_ref[...]
    y = jax.lax.dot_general(
        aT, w1te_ref[...], (((0,), (0,)), ((), ())),
        preferred_element_type=jnp.float32,
        precision=jax.lax.Precision.DEFAULT)
    h = jnp.maximum(y[:, :_D] + b1_ref[...], 0.0)
    w = 1.0 / (1.0 + jnp.exp(-y[:, _D:_D + 1]))
    out_ref[...] = h * w


def _payload(ea_t, w1te, b1, half):
    grid = (_HALF_E // _EB,)
    blk_off = half * (_HALF_E // _EB)
    return pl.pallas_call(
        _payload_body,
        grid=grid,
        in_specs=[
            pl.BlockSpec((4, _EB), lambda i: (0, i + blk_off)),
            pl.BlockSpec((4, _DW), lambda i: (0, 0)),
            pl.BlockSpec((1, _D), lambda i: (0, 0)),
        ],
        out_specs=pl.BlockSpec((_EB, _D), lambda i: (i, 0)),
        out_shape=jax.ShapeDtypeStruct((_HALF_E, _D), jnp.float32),
    )(ea_t, w1te, b1[None, :])


def _sigmoid16(v):
    return 1.0 / (1.0 + jnp.exp(-v))


def _sc_body(half, g_hbm, idx_hbm, ea2_hbm, zeros_hbm, zeros1_hbm,
             outa_hbm, outs_hbm,
             rows0, rows1, idx0, idx1, ea0, ea1, s_v, acc_sh, sem0, sem1):
    c = lax.axis_index("c")
    s = lax.axis_index("s")
    wid = s * _NC + c
    r0 = s * _RPT
    rr = _NS * _RPT
    pltpu.sync_copy(zeros_hbm.at[pl.ds(r0, _RPT)], acc_sh.at[pl.ds(r0, _RPT)])

    @pl.when(s == _NS - 1)
    def _():
        pltpu.sync_copy(zeros_hbm.at[pl.ds(rr, _NACC - rr)],
                        acc_sh.at[pl.ds(rr, _NACC - rr)])

    pltpu.sync_copy(zeros1_hbm, s_v)

    gbase = wid * _EPT
    ibase = half * _HALF_E + gbase
    bufs = (rows0, rows1)
    idxs = (idx0, idx1)
    eas = (ea0, ea1)
    sems = (sem0, sem1)

    def start(off, slot):
        pltpu.async_copy(g_hbm.at[pl.ds(gbase + off, _CH)],
                         bufs[slot], sems[slot])
        pltpu.async_copy(idx_hbm.at[pl.ds(ibase + off, _CH)],
                         idxs[slot], sems[slot])
        pltpu.async_copy(ea2_hbm.at[pl.ds(ibase + off, _CH)],
                         eas[slot], sems[slot])

    start(0, 0)

    def outer(gi, carry):
        for b in range(2):
            j = gi * 2 + b

            @pl.when(j + 1 < _CPT)
            def _():
                start((j + 1) * _CH, 1 - b)

            pltpu.make_async_copy(
                g_hbm.at[pl.ds(0, _CH)], bufs[b], sems[b]).wait()
            pltpu.make_async_copy(
                idx_hbm.at[pl.ds(0, _CH)], idxs[b], sems[b]).wait()
            pltpu.make_async_copy(
                ea2_hbm.at[pl.ds(0, _CH)], eas[b], sems[b]).wait()
            pltpu.sync_copy(bufs[b], acc_sh.at[idxs[b]], add=True)
            for k in range(_CH // _L):
                idx16 = idxs[b][pl.ds(k * _L, _L)]
                wv = _sigmoid16(eas[b][pl.ds(k * _L, _L)])
                plsc.addupdate_scatter(s_v, [idx16], wv)
        return carry

    lax.fori_loop(0, _CPT // 2, outer, 0)

    plsc.subcore_barrier()
    pltpu.sync_copy(acc_sh.at[pl.ds(r0, _RPT)],
                    outa_hbm.at[c, pl.ds(r0, _RPT)])

    @pl.when(s == _NS - 1)
    def _():
        pltpu.sync_copy(acc_sh.at[pl.ds(rr, _N_NODES - rr)],
                        outa_hbm.at[c, pl.ds(rr, _N_NODES - rr)])

    pltpu.sync_copy(s_v, outs_hbm.at[wid])


def _sc_scatter(payload, idx1d, ea2_1d, zeros, zeros1, half):
    mesh = plsc.VectorSubcoreMesh(core_axis_name="c", subcore_axis_name="s")
    fn = functools.partial(
        pl.kernel,
        mesh=mesh,
        compiler_params=pltpu.CompilerParams(needs_layout_passes=False),
        out_type=[
            jax.ShapeDtypeStruct((_NC, _N_NODES, _D), jnp.float32),
            jax.ShapeDtypeStruct((_NW, _NACC), jnp.float32),
        ],
        scratch_types=[
            pltpu.VMEM((_CH, _D), jnp.float32),
            pltpu.VMEM((_CH, _D), jnp.float32),
            pltpu.VMEM((_CH,), jnp.int32),
            pltpu.VMEM((_CH,), jnp.int32),
            pltpu.VMEM((_CH,), jnp.float32),
            pltpu.VMEM((_CH,), jnp.float32),
            pltpu.VMEM((_NACC,), jnp.float32),
            pltpu.VMEM_SHARED((_NACC, _D), jnp.float32),
            pltpu.SemaphoreType.DMA,
            pltpu.SemaphoreType.DMA,
        ],
    )(functools.partial(_sc_body, half))
    return fn(payload, idx1d, ea2_1d, zeros, zeros1)


_NB = 1000


def _final_body(x_ref, acca_ref, accb_ref, sa_ref, sb_ref,
                w2_ref, b2_ref, out_ref):
    a = (acca_ref[0] + acca_ref[1]) + (accb_ref[0] + accb_ref[1])
    y = jax.lax.dot_general(
        a, w2_ref[...], (((1,), (1,)), ((), ())),
        preferred_element_type=jnp.float32,
        precision=jax.lax.Precision.HIGHEST)
    sw = (jnp.sum(sa_ref[...], axis=1, keepdims=True)
          + jnp.sum(sb_ref[...], axis=1, keepdims=True))
    out_ref[...] = x_ref[...] + y + sw * b2_ref[...]


def _final(x, acc_a, acc_b, sa_t, sb_t, W2, b2):
    grid = (_N_NODES // _NB,)
    return pl.pallas_call(
        _final_body,
        grid=grid,
        in_specs=[
            pl.BlockSpec((_NB, _D), lambda i: (i, 0)),
            pl.BlockSpec((_NC, _NB, _D), lambda i: (0, i, 0)),
            pl.BlockSpec((_NC, _NB, _D), lambda i: (0, i, 0)),
            pl.BlockSpec((_NB, _NW), lambda i: (i, 0)),
            pl.BlockSpec((_NB, _NW), lambda i: (i, 0)),
            pl.BlockSpec((_D, _D), lambda i: (0, 0)),
            pl.BlockSpec((1, _D), lambda i: (0, 0)),
        ],
        out_specs=pl.BlockSpec((_NB, _D), lambda i: (i, 0)),
        out_shape=jax.ShapeDtypeStruct((_N_NODES, _D), jnp.float32),
    )(x, acc_a, acc_b, sa_t, sb_t, W2, b2[None, :])


def kernel(x, edge_index, edge_attr, W1, b1, W2, b2):
    npad = _EPAD - _N_EDGES
    dst = edge_index[1].astype(jnp.int32)
    pad_idx = _N_NODES + (jnp.arange(npad, dtype=jnp.int32) % 16)
    idx1d = jnp.concatenate([dst, pad_idx])
    ea_t = jnp.pad(edge_attr.T, ((0, 0), (0, npad)))
    ea2_1d = ea_t[2]
    e2 = jnp.zeros((4, _DW - _D), jnp.float32).at[2, 0].set(1.0)
    w1te = jnp.concatenate([W1.T, e2], axis=1)
    zeros = jnp.zeros((_NACC, _D), jnp.float32)
    zeros1 = jnp.zeros((_NACC,), jnp.float32)

    pay_a = _payload(ea_t, w1te, b1, 0)
    pay_b = _payload(ea_t, w1te, b1, 1)
    acc_a, s_a = _sc_scatter(pay_a, idx1d, ea2_1d, zeros, zeros1, 0)
    acc_b, s_b = _sc_scatter(pay_b, idx1d, ea2_1d, zeros, zeros1, 1)
    return _final(x, acc_a, acc_b, s_a.T, s_b.T, W2, b2)


# speedup vs baseline: 1.0494x; 1.0494x over previous
import functools

import jax
import jax.numpy as jnp
from jax import lax
from jax.experimental import pallas as pl
from jax.experimental.pallas import tpu as pltpu
from jax.experimental.pallas import tpu_sc as plsc

_N_NODES = 10000
_N_EDGES = 320000
_D = 128

_NC, _NS = 2, 16
_NW = _NC * _NS
_CH = 128
_EPAD = 327680
_NHALF = 2
_HALF_E = _EPAD // _NHALF
_CPT = _HALF_E // (_NW * _CH)
_EPT = _CPT * _CH
_NACC = _N_NODES + 16
_RPT = 624
_L = 16


_EB = 10240


_DW = _D + 16


def _payload_body(eat_ref, w1te_ref, b1_ref, out_ref):
    aT = eat_ref[...]
    y = jax.lax.dot_general(
        aT, w1te_ref[...], (((0,), (0,)), ((), ())),
        preferred_element_type=jnp.float32,
        precision=jax.lax.Precision.DEFAULT)
    h = jnp.maximum(y[:, :_D] + b1_ref[...], 0.0)
    w = 1.0 / (1.0 + jnp.exp(-y[:, _D:_D + 1]))
    out_ref[...] = h * w


def _payload(ea_t, w1te, b1, half):
    grid = (_HALF_E // _EB,)
    blk_off = half * (_HALF_E // _EB)
    return pl.pallas_call(
        _payload_body,
        grid=grid,
        in_specs=[
            pl.BlockSpec((4, _EB), lambda i: (0, i + blk_off)),
            pl.BlockSpec((4, _DW), lambda i: (0, 0)),
            pl.BlockSpec((1, _D), lambda i: (0, 0)),
        ],
        out_specs=pl.BlockSpec((_EB, _D), lambda i: (i, 0)),
        out_shape=jax.ShapeDtypeStruct((_HALF_E, _D), jnp.float32),
    )(ea_t, w1te, b1[None, :])


def _sigmoid16(v):
    return 1.0 / (1.0 + jnp.exp(-v))


def _sc_body(g_hbm, idx_hbm, ea2_hbm, zeros_hbm, outa_hbm, outs_hbm,
             rows0, rows1, idx0, idx1, ea0, ea1, s_v, acc_sh, sem0, sem1):
    c = lax.axis_index("c")
    s = lax.axis_index("s")
    wid = s * _NC + c
    r0 = s * _RPT
    rr = _NS * _RPT
    pltpu.sync_copy(zeros_hbm.at[pl.ds(r0, _RPT)], acc_sh.at[pl.ds(r0, _RPT)])

    @pl.when(s == _NS - 1)
    def _():
        pltpu.sync_copy(zeros_hbm.at[pl.ds(rr, _NACC - rr)],
                        acc_sh.at[pl.ds(rr, _NACC - rr)])

    def zbody(i, carry):
        s_v[pl.ds(i * _L, _L)] = jnp.zeros((_L,), jnp.float32)
        return carry

    lax.fori_loop(0, _NACC // _L, zbody, 0)

    ebase = wid * _EPT
    bufs = (rows0, rows1)
    idxs = (idx0, idx1)
    eas = (ea0, ea1)
    sems = (sem0, sem1)

    def start(off, slot):
        pltpu.async_copy(g_hbm.at[pl.ds(off, _CH)], bufs[slot], sems[slot])
        pltpu.async_copy(idx_hbm.at[pl.ds(off, _CH)], idxs[slot], sems[slot])
        pltpu.async_copy(ea2_hbm.at[pl.ds(off, _CH)], eas[slot], sems[slot])

    start(ebase, 0)

    def outer(gi, carry):
        for b in range(2):
            j = gi * 2 + b

            @pl.when(j + 1 < _CPT)
            def _():
                start(ebase + (j + 1) * _CH, 1 - b)

            pltpu.make_async_copy(
                g_hbm.at[pl.ds(0, _CH)], bufs[b], sems[b]).wait()
            pltpu.make_async_copy(
                idx_hbm.at[pl.ds(0, _CH)], idxs[b], sems[b]).wait()
            pltpu.make_async_copy(
                ea2_hbm.at[pl.ds(0, _CH)], eas[b], sems[b]).wait()
            pltpu.sync_copy(bufs[b], acc_sh.at[idxs[b]], add=True)
            for k in range(_CH // _L):
                idx16 = idxs[b][pl.ds(k * _L, _L)]
                wv = _sigmoid16(eas[b][pl.ds(k * _L, _L)])
                plsc.addupdate_scatter(s_v, [idx16], wv)
        return carry

    lax.fori_loop(0, _CPT // 2, outer, 0)

    plsc.subcore_barrier()
    pltpu.sync_copy(acc_sh.at[pl.ds(r0, _RPT)],
                    outa_hbm.at[c, pl.ds(r0, _RPT)])

    @pl.when(s == _NS - 1)
    def _():
        pltpu.sync_copy(acc_sh.at[pl.ds(rr, _N_NODES - rr)],
                        outa_hbm.at[c, pl.ds(rr, _N_NODES - rr)])

    pltpu.sync_copy(s_v, outs_hbm.at[wid])


def _sc_scatter(payload, idx1d, ea2_1d, zeros):
    mesh = plsc.VectorSubcoreMesh(core_axis_name="c", subcore_axis_name="s")
    fn = functools.partial(
        pl.kernel,
        mesh=mesh,
        compiler_params=pltpu.CompilerParams(needs_layout_passes=False),
        out_type=[
            jax.ShapeDtypeStruct((_NC, _N_NODES, _D), jnp.float32),
            jax.ShapeDtypeStruct((_NW, _NACC), jnp.float32),
        ],
        scratch_types=[
            pltpu.VMEM((_CH, _D), jnp.float32),
            pltpu.VMEM((_CH, _D), jnp.float32),
            pltpu.VMEM((_CH,), jnp.int32),
            pltpu.VMEM((_CH,), jnp.int32),
            pltpu.VMEM((_CH,), jnp.float32),
            pltpu.VMEM((_CH,), jnp.float32),
            pltpu.VMEM((_NACC,), jnp.float32),
            pltpu.VMEM_SHARED((_NACC, _D), jnp.float32),
            pltpu.SemaphoreType.DMA,
            pltpu.SemaphoreType.DMA,
        ],
    )(_sc_body)
    return fn(payload, idx1d, ea2_1d, zeros)


_NB = 1000


def _final_body(x_ref, acca_ref, accb_ref, sa_ref, sb_ref,
                w2_ref, b2_ref, out_ref):
    a = (acca_ref[0] + acca_ref[1]) + (accb_ref[0] + accb_ref[1])
    y = jax.lax.dot_general(
        a, w2_ref[...], (((1,), (1,)), ((), ())),
        preferred_element_type=jnp.float32,
        precision=jax.lax.Precision.HIGHEST)
    sw = (jnp.sum(sa_ref[...], axis=1, keepdims=True)
          + jnp.sum(sb_ref[...], axis=1, keepdims=True))
    out_ref[...] = x_ref[...] + y + sw * b2_ref[...]


def _final(x, acc_a, acc_b, sa_t, sb_t, W2, b2):
    grid = (_N_NODES // _NB,)
    return pl.pallas_call(
        _final_body,
        grid=grid,
        in_specs=[
            pl.BlockSpec((_NB, _D), lambda i: (i, 0)),
            pl.BlockSpec((_NC, _NB, _D), lambda i: (0, i, 0)),
            pl.BlockSpec((_NC, _NB, _D), lambda i: (0, i, 0)),
            pl.BlockSpec((_NB, _NW), lambda i: (i, 0)),
            pl.BlockSpec((_NB, _NW), lambda i: (i, 0)),
            pl.BlockSpec((_D, _D), lambda i: (0, 0)),
            pl.BlockSpec((1, _D), lambda i: (0, 0)),
        ],
        out_specs=pl.BlockSpec((_NB, _D), lambda i: (i, 0)),
        out_shape=jax.ShapeDtypeStruct((_N_NODES, _D), jnp.float32),
    )(x, acc_a, acc_b, sa_t, sb_t, W2, b2[None, :])


def kernel(x, edge_index, edge_attr, W1, b1, W2, b2):
    npad = _EPAD - _N_EDGES
    dst = edge_index[1].astype(jnp.int32)
    pad_idx = _N_NODES + (jnp.arange(npad, dtype=jnp.int32) % 16)
    idx1d = jnp.concatenate([dst, pad_idx])
    ea_t = jnp.pad(edge_attr.T, ((0, 0), (0, npad)))
    ea2_1d = ea_t[2]
    e2 = jnp.zeros((4, _DW - _D), jnp.float32).at[2, 0].set(1.0)
    w1te = jnp.concatenate([W1.T, e2], axis=1)
    zeros = jnp.zeros((_NACC, _D), jnp.float32)

    pay_a = _payload(ea_t, w1te, b1, 0)
    pay_b = _payload(ea_t, w1te, b1, 1)
    acc_a, s_a = _sc_scatter(pay_a, idx1d[:_HALF_E], ea2_1d[:_HALF_E], zeros)
    acc_b, s_b = _sc_scatter(pay_b, idx1d[_HALF_E:], ea2_1d[_HALF_E:], zeros)
    return _final(x, acc_a, acc_b, s_a.T, s_b.T, W2, b2)


# speedup vs baseline: 1.0879x; 1.0366x over previous
import functools

import jax
import jax.numpy as jnp
from jax import lax
from jax.experimental import pallas as pl
from jax.experimental.pallas import tpu as pltpu
from jax.experimental.pallas import tpu_sc as plsc

_N_NODES = 10000
_N_EDGES = 320000
_D = 128

_NC, _NS = 2, 16
_NW = _NC * _NS
_CH = 128
_EPAD = 327680
_NHALF = 2
_HALF_E = _EPAD // _NHALF
_CPT = _HALF_E // (_NW * _CH)
_EPT = _CPT * _CH
_NACC = _N_NODES + 16
_RPT = 624
_L = 16


_EB = 10240


_DW = _D + 16


def _payload_body(eat_ref, w1te_ref, out_ref):
    aT = eat_ref[...]
    y = jax.lax.dot_general(
        aT, w1te_ref[...], (((0,), (0,)), ((), ())),
        preferred_element_type=jnp.float32,
        precision=jax.lax.Precision.DEFAULT)
    h = jnp.maximum(y[:, :_D], 0.0)
    w = 1.0 / (1.0 + jnp.exp(-y[:, _D:_D + 1]))
    out_ref[...] = h * w


def _payload(ea_t, w1te, half):
    grid = (_HALF_E // _EB,)
    blk_off = half * (_HALF_E // _EB)
    return pl.pallas_call(
        _payload_body,
        grid=grid,
        in_specs=[
            pl.BlockSpec((5, _EB), lambda i: (0, i + blk_off)),
            pl.BlockSpec((5, _DW), lambda i: (0, 0)),
        ],
        out_specs=pl.BlockSpec((_EB, _D), lambda i: (i, 0)),
        out_shape=jax.ShapeDtypeStruct((_HALF_E, _D), jnp.float32),
    )(ea_t, w1te)


def _sigmoid16(v):
    return 1.0 / (1.0 + jnp.exp(-v))


def _sc_body(g_hbm, idx_hbm, ea2_hbm, outa_hbm, outs_hbm,
             rows0, rows1, idx0, idx1, ea0, ea1, s_v, acc_sh, sem0, sem1):
    c = lax.axis_index("c")
    s = lax.axis_index("s")
    wid = s * _NC + c
    r0 = s * _RPT
    rr = _NS * _RPT
    zv = jnp.zeros((_L,), jnp.float32)

    def zbody(i, carry):
        for k in range(_D // _L):
            rows0[i, pl.ds(k * _L, _L)] = zv
            rows1[i, pl.ds(k * _L, _L)] = zv
        return carry

    lax.fori_loop(0, _CH, zbody, 0)
    pltpu.async_copy(rows0, acc_sh.at[pl.ds(r0, _CH)], sem0)
    pltpu.async_copy(rows1, acc_sh.at[pl.ds(r0 + _CH, _CH)], sem1)
    pltpu.async_copy(rows0, acc_sh.at[pl.ds(r0 + 2 * _CH, _CH)], sem0)
    pltpu.async_copy(rows1, acc_sh.at[pl.ds(r0 + 3 * _CH, _CH)], sem1)
    pltpu.async_copy(rows0.at[pl.ds(0, _RPT - 4 * _CH)],
                     acc_sh.at[pl.ds(r0 + 4 * _CH, _RPT - 4 * _CH)], sem0)

    @pl.when(s == _NS - 1)
    def _():
        pltpu.async_copy(rows1.at[pl.ds(0, _NACC - rr)],
                         acc_sh.at[pl.ds(rr, _NACC - rr)], sem1)

    def sbody(i, carry):
        for k in range(8):
            s_v[pl.ds(i * 8 * _L + k * _L, _L)] = zv
        return carry

    lax.fori_loop(0, _NACC // (8 * _L), sbody, 0)
    for k in range(2):
        s_v[pl.ds((_NACC // (8 * _L)) * 8 * _L + k * _L, _L)] = zv

    pltpu.make_async_copy(rows0, acc_sh.at[pl.ds(r0, _CH)], sem0).wait()
    pltpu.make_async_copy(rows0, acc_sh.at[pl.ds(r0, _CH)], sem0).wait()
    pltpu.make_async_copy(rows0.at[pl.ds(0, _RPT - 4 * _CH)],
                          acc_sh.at[pl.ds(r0, _RPT - 4 * _CH)], sem0).wait()
    pltpu.make_async_copy(rows1, acc_sh.at[pl.ds(r0, _CH)], sem1).wait()
    pltpu.make_async_copy(rows1, acc_sh.at[pl.ds(r0, _CH)], sem1).wait()

    @pl.when(s == _NS - 1)
    def _():
        pltpu.make_async_copy(rows1.at[pl.ds(0, _NACC - rr)],
                              acc_sh.at[pl.ds(rr, _NACC - rr)], sem1).wait()

    ebase = wid * _EPT
    bufs = (rows0, rows1)
    idxs = (idx0, idx1)
    eas = (ea0, ea1)
    sems = (sem0, sem1)

    def start(off, slot):
        pltpu.async_copy(g_hbm.at[pl.ds(off, _CH)], bufs[slot], sems[slot])
        pltpu.async_copy(idx_hbm.at[pl.ds(off, _CH)], idxs[slot], sems[slot])
        pltpu.async_copy(ea2_hbm.at[pl.ds(off, _CH)], eas[slot], sems[slot])

    start(ebase, 0)

    def outer(gi, carry):
        for b in range(2):
            j = gi * 2 + b

            @pl.when(j + 1 < _CPT)
            def _():
                start(ebase + (j + 1) * _CH, 1 - b)

            pltpu.make_async_copy(
                g_hbm.at[pl.ds(0, _CH)], bufs[b], sems[b]).wait()
            pltpu.make_async_copy(
                idx_hbm.at[pl.ds(0, _CH)], idxs[b], sems[b]).wait()
            pltpu.make_async_copy(
                ea2_hbm.at[pl.ds(0, _CH)], eas[b], sems[b]).wait()
            pltpu.sync_copy(bufs[b], acc_sh.at[idxs[b]], add=True)
            for k in range(_CH // _L):
                idx16 = idxs[b][pl.ds(k * _L, _L)]
                wv = _sigmoid16(eas[b][pl.ds(k * _L, _L)])
                plsc.addupdate_scatter(s_v, [idx16], wv)
        return carry

    lax.fori_loop(0, _CPT // 2, outer, 0)

    plsc.subcore_barrier()
    pltpu.sync_copy(acc_sh.at[pl.ds(r0, _RPT)],
                    outa_hbm.at[c, pl.ds(r0, _RPT)])

    @pl.when(s == _NS - 1)
    def _():
        pltpu.sync_copy(acc_sh.at[pl.ds(rr, _N_NODES - rr)],
                        outa_hbm.at[c, pl.ds(rr, _N_NODES - rr)])

    pltpu.sync_copy(s_v, outs_hbm.at[wid])


def _sc_scatter(payload, idx1d, ea2_1d):
    mesh = plsc.VectorSubcoreMesh(core_axis_name="c", subcore_axis_name="s")
    fn = functools.partial(
        pl.kernel,
        mesh=mesh,
        compiler_params=pltpu.CompilerParams(needs_layout_passes=False),
        out_type=[
            jax.ShapeDtypeStruct((_NC, _N_NODES, _D), jnp.float32),
            jax.ShapeDtypeStruct((_NW, _NACC), jnp.float32),
        ],
        scratch_types=[
            pltpu.VMEM((_CH, _D), jnp.float32),
            pltpu.VMEM((_CH, _D), jnp.float32),
            pltpu.VMEM((_CH,), jnp.int32),
            pltpu.VMEM((_CH,), jnp.int32),
            pltpu.VMEM((_CH,), jnp.float32),
            pltpu.VMEM((_CH,), jnp.float32),
            pltpu.VMEM((_NACC,), jnp.float32),
            pltpu.VMEM_SHARED((_NACC, _D), jnp.float32),
            pltpu.SemaphoreType.DMA,
            pltpu.SemaphoreType.DMA,
        ],
    )(_sc_body)
    return fn(payload, idx1d, ea2_1d)


_NB = 1000


def _final_body(x_ref, acca_ref, accb_ref, sa_ref, sb_ref,
                w2_ref, b2_ref, out_ref):
    a = (acca_ref[0] + acca_ref[1]) + (accb_ref[0] + accb_ref[1])
    y = jax.lax.dot_general(
        a, w2_ref[...], (((1,), (1,)), ((), ())),
        preferred_element_type=jnp.float32,
        precision=jax.lax.Precision.HIGHEST)
    sw = (jnp.sum(sa_ref[...], axis=1, keepdims=True)
          + jnp.sum(sb_ref[...], axis=1, keepdims=True))
    out_ref[...] = x_ref[...] + y + sw * b2_ref[...]


def _final(x, acc_a, acc_b, sa_t, sb_t, W2, b2):
    grid = (_N_NODES // _NB,)
    return pl.pallas_call(
        _final_body,
        grid=grid,
        in_specs=[
            pl.BlockSpec((_NB, _D), lambda i: (i, 0)),
            pl.BlockSpec((_NC, _NB, _D), lambda i: (0, i, 0)),
            pl.BlockSpec((_NC, _NB, _D), lambda i: (0, i, 0)),
            pl.BlockSpec((_NB, _NW), lambda i: (i, 0)),
            pl.BlockSpec((_NB, _NW), lambda i: (i, 0)),
            pl.BlockSpec((_D, _D), lambda i: (0, 0)),
            pl.BlockSpec((1, _D), lambda i: (0, 0)),
        ],
        out_specs=pl.BlockSpec((_NB, _D), lambda i: (i, 0)),
        out_shape=jax.ShapeDtypeStruct((_N_NODES, _D), jnp.float32),
    )(x, acc_a, acc_b, sa_t, sb_t, W2, b2[None, :])


def kernel(x, edge_index, edge_attr, W1, b1, W2, b2):
    npad = _EPAD - _N_EDGES
    dst = edge_index[1].astype(jnp.int32)
    pad_idx = _N_NODES + (jnp.arange(npad, dtype=jnp.int32) % 16)
    idx1d = jnp.concatenate([dst, pad_idx])
    ea_t = jnp.pad(edge_attr.T, ((0, 1), (0, npad)), constant_values=1.0)
    ea2_1d = ea_t[2]
    e2 = jnp.zeros((4, _DW - _D), jnp.float32).at[2, 0].set(1.0)
    top = jnp.concatenate([W1.T, e2], axis=1)
    bot = jnp.concatenate(
        [b1[None, :], jnp.zeros((1, _DW - _D), jnp.float32)], axis=1)
    w1te = jnp.concatenate([top, bot], axis=0)

    pay_a = _payload(ea_t, w1te, 0)
    pay_b = _payload(ea_t, w1te, 1)
    acc_a, s_a = _sc_scatter(pay_a, idx1d[:_HALF_E], ea2_1d[:_HALF_E])
    acc_b, s_b = _sc_scatter(pay_b, idx1d[_HALF_E:], ea2_1d[_HALF_E:])
    return _final(x, acc_a, acc_b, s_a.T, s_b.T, W2, b2)
